# baseline (speedup 1.0000x reference)
import functools

import jax
import jax.numpy as jnp
from jax import lax
from jax.experimental import pallas as pl
from jax.experimental.pallas import tpu as pltpu
from jax.experimental.pallas import tpu_sc as plsc

N_NODES = 50000
E = 800000
R = 8
C = 16
LIMIT_A = -0.1
LIMIT_B = 1.1

NC = 2
NS = 16
NW = NC * NS

E_PAD = 819200
EDGES_PER_W = E_PAD // NW
CHUNK = 512
GROUPS = CHUNK // 128
N_CHUNKS = EDGES_PER_W // CHUNK
IDX_ROWS_PER_W = EDGES_PER_W // 128

BE = 8192
GSTEPS = E_PAD // BE
BR = BE // 8
FROWS = E_PAD * C // 128


def _sc_gather_body(table, sidx, didx, p_out, q_out,
                    sidx_all, didx_all,
                    srow0, srow1, drow0, drow1,
                    gs0, gs1, ws0, ws1):
    wid = lax.axis_index("s") * NC + lax.axis_index("c")
    row0 = wid * IDX_ROWS_PER_W
    base0 = wid * EDGES_PER_W
    srow_v = (srow0, srow1)
    drow_v = (drow0, drow1)
    gsem = (gs0, gs1)
    wsem = (ws0, ws1)

    pltpu.sync_copy(sidx.at[pl.ds(row0, IDX_ROWS_PER_W)], sidx_all)
    pltpu.sync_copy(didx.at[pl.ds(row0, IDX_ROWS_PER_W)], didx_all)

    def launch_g(i, b):
        for g in range(GROUPS):
            pltpu.async_copy(table.at[sidx_all.at[i * GROUPS + g]],
                             srow_v[b].at[pl.ds(g * 128, 128)], gsem[b])
            pltpu.async_copy(table.at[didx_all.at[i * GROUPS + g]],
                             drow_v[b].at[pl.ds(g * 128, 128)], gsem[b])

    def wait_g(b):
        pltpu.make_async_copy(p_out.at[pl.ds(0, CHUNK)], srow_v[b], gsem[b]).wait()
        pltpu.make_async_copy(p_out.at[pl.ds(0, CHUNK)], drow_v[b], gsem[b]).wait()

    def launch_wb(i, b):
        off = base0 + i * CHUNK
        pltpu.async_copy(srow_v[b], p_out.at[pl.ds(off, CHUNK)], wsem[b])
        pltpu.async_copy(drow_v[b], q_out.at[pl.ds(off, CHUNK)], wsem[b])

    def wait_wb(b):
        pltpu.make_async_copy(p_out.at[pl.ds(0, CHUNK)], srow_v[b], wsem[b]).wait()
        pltpu.make_async_copy(p_out.at[pl.ds(0, CHUNK)], drow_v[b], wsem[b]).wait()

    launch_g(0, 0)
    launch_g(1, 1)

    def body(k, carry):
        for b in (0, 1):
            i = 2 * k + b
            wait_g(b)
            launch_wb(i, b)
            wait_wb(b)
            launch_g(i + 2, b)
        return carry

    lax.fori_loop(0, N_CHUNKS // 2 - 1, body, 0)

    for b in (0, 1):
        wait_g(b)
        launch_wb(N_CHUNKS - 2 + b, b)
    wait_wb(0)
    wait_wb(1)


def _sc_gather(assignments, src_idx, dst_idx):
    kern = pl.kernel(
        _sc_gather_body,
        out_type=(jax.ShapeDtypeStruct((E_PAD, C), jnp.float32),
                  jax.ShapeDtypeStruct((E_PAD, C), jnp.float32)),
        mesh=plsc.VectorSubcoreMesh(core_axis_name="c", subcore_axis_name="s"),
        scratch_types=[
            pltpu.VMEM((IDX_ROWS_PER_W, 128), jnp.int32),
            pltpu.VMEM((IDX_ROWS_PER_W, 128), jnp.int32),
            pltpu.VMEM((CHUNK, C), jnp.float32),
            pltpu.VMEM((CHUNK, C), jnp.float32),
            pltpu.VMEM((CHUNK, C), jnp.float32),
            pltpu.VMEM((CHUNK, C), jnp.float32),
            pltpu.SemaphoreType.DMA,
            pltpu.SemaphoreType.DMA,
            pltpu.SemaphoreType.DMA,
            pltpu.SemaphoreType.DMA,
        ],
        compiler_params=pltpu.CompilerParams(use_tc_tiling_on_sc=False),
    )
    return kern(assignments, src_idx, dst_idx)


def _tc_loss_body(p_ref, q_ref, et_ref, il_ref, la_ref, b_ref, out_ref):
    step = pl.program_id(0)

    z = 1.0 / (1.0 + jnp.exp(-la_ref[...]))
    z = jnp.clip(z * (LIMIT_B - LIMIT_A) + LIMIT_A, 0.0, 1.0)
    w2 = ((1.0 / (1.0 + jnp.exp(-il_ref[...]))) * z)
    w2 = w2.astype(jnp.bfloat16)

    pT = jnp.transpose(p_ref[...]).astype(jnp.bfloat16)
    qT = jnp.transpose(q_ref[...]).astype(jnp.bfloat16)

    ki = lax.broadcasted_iota(jnp.int32, (R, R * C), 0)
    kj = lax.broadcasted_iota(jnp.int32, (R, R * C), 1)
    kred = (ki == kj // C).astype(jnp.bfloat16)
    sub8 = lax.broadcasted_iota(jnp.int32, (R, BR), 0)
    row_iota = lax.broadcasted_iota(jnp.int32, (1, BR), 1)
    valid = (step * BR + row_iota) < (E // 8)

    tot = jnp.zeros((), jnp.float32)
    for g in range(8):
        qtg = qT[g * C:(g + 1) * C, :]
        ptg = pT[g * C:(g + 1) * C, :]
        t2 = lax.dot_general(w2, qtg, (((1,), (0,)), ((), ())),
                             preferred_element_type=jnp.float32
                             ).astype(jnp.bfloat16)
        ptile = jnp.concatenate([ptg] * R, axis=0)
        u2 = t2 * ptile
        l8t = lax.dot_general(kred, u2, (((1,), (0,)), ((), ())),
                              preferred_element_type=jnp.float32)

        et_row = et_ref[0, g:g + 1, :]
        onehot = sub8 == et_row
        lsel = jnp.where(onehot, l8t, 0.0)
        bsel = jnp.where(onehot, b_ref[...], 0.0)
        logits = (jnp.sum(lsel, axis=0, keepdims=True)
                  + jnp.sum(bsel, axis=0, keepdims=True))

        sp = jnp.maximum(-logits, 0.0) + jnp.log(1.0 + jnp.exp(-jnp.abs(logits)))
        sp = jnp.where(valid, sp, 0.0)
        tot = tot + jnp.sum(sp)

    @pl.when(step == 0)
    def _init():
        out_ref[...] = jnp.zeros_like(out_ref)

    out_ref[...] = out_ref[...] + tot.reshape(1, 1)

    @pl.when(step == GSTEPS - 1)
    def _final():
        out_ref[...] = out_ref[...] * (1.0 / E)


def _tc_loss(p, q, et2, il2, la2, bias2):
    return pl.pallas_call(
        _tc_loss_body,
        grid=(GSTEPS,),
        in_specs=[
            pl.BlockSpec((BR, 128), lambda i: (i, 0)),
            pl.BlockSpec((BR, 128), lambda i: (i, 0)),
            pl.BlockSpec((1, R, BR), lambda i: (i, 0, 0)),
            pl.BlockSpec((R * C, C), lambda i: (0, 0)),
            pl.BlockSpec((R * C, C), lambda i: (0, 0)),
            pl.BlockSpec((R, 1), lambda i: (0, 0)),
        ],
        out_specs=pl.BlockSpec((1, 1), lambda i: (0, 0)),
        out_shape=jax.ShapeDtypeStruct((1, 1), jnp.float32),
        compiler_params=pltpu.CompilerParams(
            dimension_semantics=("arbitrary",)),
    )(p, q, et2, il2, la2, bias2)


def kernel(assignments, edge_index, edge_type, inter_cluster_logits,
           absent_bias, log_alpha):
    pad = E_PAD - E
    zpad = jnp.zeros((pad,), jnp.int32)
    src_p = jnp.concatenate([edge_index[0], zpad]).reshape(E_PAD // 128, 128)
    dst_p = jnp.concatenate([edge_index[1], zpad]).reshape(E_PAD // 128, 128)
    et2 = (jnp.concatenate([edge_type, zpad])
           .reshape(GSTEPS, BR, 8).transpose(0, 2, 1))
    il2 = inter_cluster_logits.reshape(R * C, C)
    la2 = log_alpha.reshape(R * C, C)
    bias2 = absent_bias.reshape(R, 1)

    p, q = _sc_gather(assignments, src_p, dst_p)
    pf = p.reshape(FROWS, 128)
    qf = q.reshape(FROWS, 128)
    out = _tc_loss(pf, qf, et2, il2, la2, bias2)
    return out[0, 0]


# speedup vs baseline: 1.1189x; 1.1189x over previous
import functools

import jax
import jax.numpy as jnp
from jax import lax
from jax.experimental import pallas as pl
from jax.experimental.pallas import tpu as pltpu
from jax.experimental.pallas import tpu_sc as plsc

N_NODES = 50000
E = 800000
R = 8
C = 16
LIMIT_A = -0.1
LIMIT_B = 1.1

NC = 2
NS = 16
NW = NC * NS

E_PAD = 819200
EDGES_PER_W = E_PAD // NW
CHUNK = 512
GROUPS = CHUNK // 128
N_CHUNKS = EDGES_PER_W // CHUNK
IDX_ROWS_PER_W = EDGES_PER_W // 128

BE = 16384
GSTEPS = E_PAD // BE
BR = BE // 8
FROWS = E_PAD * C // 128


def _sc_gather_body(table, sidx, didx, p_out, q_out,
                    sidx_all, didx_all,
                    srow0, srow1, drow0, drow1,
                    gs0, gs1, ws0, ws1):
    wid = lax.axis_index("s") * NC + lax.axis_index("c")
    row0 = wid * IDX_ROWS_PER_W
    base0 = wid * EDGES_PER_W
    srow_v = (srow0, srow1)
    drow_v = (drow0, drow1)
    gsem = (gs0, gs1)
    wsem = (ws0, ws1)

    pltpu.sync_copy(sidx.at[pl.ds(row0, IDX_ROWS_PER_W)], sidx_all)
    pltpu.sync_copy(didx.at[pl.ds(row0, IDX_ROWS_PER_W)], didx_all)

    def launch_g(i, b):
        for g in range(GROUPS):
            pltpu.async_copy(table.at[sidx_all.at[i * GROUPS + g]],
                             srow_v[b].at[pl.ds(g * 128, 128)], gsem[b])
            pltpu.async_copy(table.at[didx_all.at[i * GROUPS + g]],
                             drow_v[b].at[pl.ds(g * 128, 128)], gsem[b])

    def wait_g(b):
        pltpu.make_async_copy(p_out.at[pl.ds(0, CHUNK)], srow_v[b], gsem[b]).wait()
        pltpu.make_async_copy(p_out.at[pl.ds(0, CHUNK)], drow_v[b], gsem[b]).wait()

    def launch_wb(i, b):
        off = base0 + i * CHUNK
        pltpu.async_copy(srow_v[b], p_out.at[pl.ds(off, CHUNK)], wsem[b])
        pltpu.async_copy(drow_v[b], q_out.at[pl.ds(off, CHUNK)], wsem[b])

    def wait_wb(b):
        pltpu.make_async_copy(p_out.at[pl.ds(0, CHUNK)], srow_v[b], wsem[b]).wait()
        pltpu.make_async_copy(p_out.at[pl.ds(0, CHUNK)], drow_v[b], wsem[b]).wait()

    launch_g(0, 0)
    launch_g(1, 1)

    def body(k, carry):
        for b in (0, 1):
            i = 2 * k + b
            wait_g(b)
            launch_wb(i, b)
            wait_wb(b)
            launch_g(i + 2, b)
        return carry

    lax.fori_loop(0, N_CHUNKS // 2 - 1, body, 0)

    for b in (0, 1):
        wait_g(b)
        launch_wb(N_CHUNKS - 2 + b, b)
    wait_wb(0)
    wait_wb(1)


def _sc_gather(assignments, src_idx, dst_idx):
    kern = pl.kernel(
        _sc_gather_body,
        out_type=(jax.ShapeDtypeStruct((E_PAD, C), jnp.float32),
                  jax.ShapeDtypeStruct((E_PAD, C), jnp.float32)),
        mesh=plsc.VectorSubcoreMesh(core_axis_name="c", subcore_axis_name="s"),
        scratch_types=[
            pltpu.VMEM((IDX_ROWS_PER_W, 128), jnp.int32),
            pltpu.VMEM((IDX_ROWS_PER_W, 128), jnp.int32),
            pltpu.VMEM((CHUNK, C), jnp.float32),
            pltpu.VMEM((CHUNK, C), jnp.float32),
            pltpu.VMEM((CHUNK, C), jnp.float32),
            pltpu.VMEM((CHUNK, C), jnp.float32),
            pltpu.SemaphoreType.DMA,
            pltpu.SemaphoreType.DMA,
            pltpu.SemaphoreType.DMA,
            pltpu.SemaphoreType.DMA,
        ],
        compiler_params=pltpu.CompilerParams(use_tc_tiling_on_sc=False),
    )
    return kern(assignments, src_idx, dst_idx)


def _tc_loss_body(p_ref, q_ref, et_ref, il_ref, la_ref, b_ref, out_ref):
    step = pl.program_id(0)

    z = 1.0 / (1.0 + jnp.exp(-la_ref[...]))
    z = jnp.clip(z * (LIMIT_B - LIMIT_A) + LIMIT_A, 0.0, 1.0)
    w2 = ((1.0 / (1.0 + jnp.exp(-il_ref[...]))) * z)
    w2 = w2.astype(jnp.bfloat16)

    pT = jnp.transpose(p_ref[...]).astype(jnp.bfloat16)
    qT = jnp.transpose(q_ref[...]).astype(jnp.bfloat16)

    ki = lax.broadcasted_iota(jnp.int32, (R, R * C), 0)
    kj = lax.broadcasted_iota(jnp.int32, (R, R * C), 1)
    kred = (ki == kj // C).astype(jnp.bfloat16)
    sub8 = lax.broadcasted_iota(jnp.int32, (R, BR), 0)
    row_iota = lax.broadcasted_iota(jnp.int32, (1, BR), 1)
    valid = (step * BR + row_iota) < (E // 8)

    tot = jnp.zeros((), jnp.float32)
    for g in range(8):
        qtg = qT[g * C:(g + 1) * C, :]
        ptg = pT[g * C:(g + 1) * C, :]
        t2 = lax.dot_general(w2, qtg, (((1,), (0,)), ((), ())),
                             preferred_element_type=jnp.float32
                             ).astype(jnp.bfloat16)
        ptile = jnp.concatenate([ptg] * R, axis=0)
        u2 = t2 * ptile
        l8t = lax.dot_general(kred, u2, (((1,), (0,)), ((), ())),
                              preferred_element_type=jnp.float32)

        et_row = et_ref[0, g:g + 1, :]
        onehot = sub8 == et_row
        lsel = jnp.where(onehot, l8t, 0.0)
        bsel = jnp.where(onehot, b_ref[...], 0.0)
        logits = (jnp.sum(lsel, axis=0, keepdims=True)
                  + jnp.sum(bsel, axis=0, keepdims=True))

        sp = jnp.maximum(-logits, 0.0) + jnp.log(1.0 + jnp.exp(-jnp.abs(logits)))
        sp = jnp.where(valid, sp, 0.0)
        tot = tot + jnp.sum(sp)

    @pl.when(step == 0)
    def _init():
        out_ref[...] = jnp.zeros_like(out_ref)

    out_ref[...] = out_ref[...] + tot.reshape(1, 1)

    @pl.when(step == GSTEPS - 1)
    def _final():
        out_ref[...] = out_ref[...] * (1.0 / E)


def _tc_loss(p, q, et2, il2, la2, bias2):
    return pl.pallas_call(
        _tc_loss_body,
        grid=(GSTEPS,),
        in_specs=[
            pl.BlockSpec((BR, 128), lambda i: (i, 0)),
            pl.BlockSpec((BR, 128), lambda i: (i, 0)),
            pl.BlockSpec((1, R, BR), lambda i: (i, 0, 0)),
            pl.BlockSpec((R * C, C), lambda i: (0, 0)),
            pl.BlockSpec((R * C, C), lambda i: (0, 0)),
            pl.BlockSpec((R, 1), lambda i: (0, 0)),
        ],
        out_specs=pl.BlockSpec((1, 1), lambda i: (0, 0)),
        out_shape=jax.ShapeDtypeStruct((1, 1), jnp.float32),
        compiler_params=pltpu.CompilerParams(
            dimension_semantics=("arbitrary",)),
    )(p, q, et2, il2, la2, bias2)


def kernel(assignments, edge_index, edge_type, inter_cluster_logits,
           absent_bias, log_alpha):
    pad = E_PAD - E
    zpad = jnp.zeros((pad,), jnp.int32)
    src_p = jnp.concatenate([edge_index[0], zpad]).reshape(E_PAD // 128, 128)
    dst_p = jnp.concatenate([edge_index[1], zpad]).reshape(E_PAD // 128, 128)
    et2 = (jnp.concatenate([edge_type, zpad])
           .reshape(GSTEPS, BR, 8).transpose(0, 2, 1))
    il2 = inter_cluster_logits.reshape(R * C, C)
    la2 = log_alpha.reshape(R * C, C)
    bias2 = absent_bias.reshape(R, 1)

    p, q = _sc_gather(assignments, src_p, dst_p)
    pf = p.reshape(FROWS, 128)
    qf = q.reshape(FROWS, 128)
    out = _tc_loss(pf, qf, et2, il2, la2, bias2)
    return out[0, 0]


# speedup vs baseline: 1.1802x; 1.0548x over previous
import functools

import jax
import jax.numpy as jnp
from jax import lax
from jax.experimental import pallas as pl
from jax.experimental.pallas import tpu as pltpu
from jax.experimental.pallas import tpu_sc as plsc

N_NODES = 50000
E = 800000
R = 8
C = 16
LIMIT_A = -0.1
LIMIT_B = 1.1

NC = 2
NS = 16
NW = NC * NS

E_PAD = 819200
EDGES_PER_W = E_PAD // NW
CHUNK = 512
GROUPS = CHUNK // 128
N_CHUNKS = EDGES_PER_W // CHUNK
IDX_ROWS_PER_W = EDGES_PER_W // 128

BE = 32768
GSTEPS = E_PAD // BE
BR = BE // 8
FROWS = E_PAD * C // 128


def _sc_gather_body(table, sidx, didx, p_out, q_out,
                    sidx_all, didx_all,
                    srow0, srow1, drow0, drow1,
                    gs0, gs1, ws0, ws1):
    wid = lax.axis_index("s") * NC + lax.axis_index("c")
    row0 = wid * IDX_ROWS_PER_W
    base0 = wid * EDGES_PER_W
    srow_v = (srow0, srow1)
    drow_v = (drow0, drow1)
    gsem = (gs0, gs1)
    wsem = (ws0, ws1)

    pltpu.sync_copy(sidx.at[pl.ds(row0, IDX_ROWS_PER_W)], sidx_all)
    pltpu.sync_copy(didx.at[pl.ds(row0, IDX_ROWS_PER_W)], didx_all)

    def launch_g(i, b):
        for g in range(GROUPS):
            pltpu.async_copy(table.at[sidx_all.at[i * GROUPS + g]],
                             srow_v[b].at[pl.ds(g * 128, 128)], gsem[b])
            pltpu.async_copy(table.at[didx_all.at[i * GROUPS + g]],
                             drow_v[b].at[pl.ds(g * 128, 128)], gsem[b])

    def wait_g(b):
        pltpu.make_async_copy(p_out.at[pl.ds(0, CHUNK)], srow_v[b], gsem[b]).wait()
        pltpu.make_async_copy(p_out.at[pl.ds(0, CHUNK)], drow_v[b], gsem[b]).wait()

    def launch_wb(i, b):
        off = base0 + i * CHUNK
        pltpu.async_copy(srow_v[b], p_out.at[pl.ds(off, CHUNK)], wsem[b])
        pltpu.async_copy(drow_v[b], q_out.at[pl.ds(off, CHUNK)], wsem[b])

    def wait_wb(b):
        pltpu.make_async_copy(p_out.at[pl.ds(0, CHUNK)], srow_v[b], wsem[b]).wait()
        pltpu.make_async_copy(p_out.at[pl.ds(0, CHUNK)], drow_v[b], wsem[b]).wait()

    launch_g(0, 0)
    launch_g(1, 1)

    def body(k, carry):
        for b in (0, 1):
            i = 2 * k + b
            wait_g(b)
            launch_wb(i, b)
            wait_wb(b)
            launch_g(i + 2, b)
        return carry

    lax.fori_loop(0, N_CHUNKS // 2 - 1, body, 0)

    for b in (0, 1):
        wait_g(b)
        launch_wb(N_CHUNKS - 2 + b, b)
    wait_wb(0)
    wait_wb(1)


def _sc_gather(assignments, src_idx, dst_idx):
    kern = pl.kernel(
        _sc_gather_body,
        out_type=(jax.ShapeDtypeStruct((E_PAD, C), jnp.float32),
                  jax.ShapeDtypeStruct((E_PAD, C), jnp.float32)),
        mesh=plsc.VectorSubcoreMesh(core_axis_name="c", subcore_axis_name="s"),
        scratch_types=[
            pltpu.VMEM((IDX_ROWS_PER_W, 128), jnp.int32),
            pltpu.VMEM((IDX_ROWS_PER_W, 128), jnp.int32),
            pltpu.VMEM((CHUNK, C), jnp.float32),
            pltpu.VMEM((CHUNK, C), jnp.float32),
            pltpu.VMEM((CHUNK, C), jnp.float32),
            pltpu.VMEM((CHUNK, C), jnp.float32),
            pltpu.SemaphoreType.DMA,
            pltpu.SemaphoreType.DMA,
            pltpu.SemaphoreType.DMA,
            pltpu.SemaphoreType.DMA,
        ],
        compiler_params=pltpu.CompilerParams(use_tc_tiling_on_sc=False),
    )
    return kern(assignments, src_idx, dst_idx)


def _tc_loss_body(p_ref, q_ref, et_ref, il_ref, la_ref, b_ref, out_ref):
    step = pl.program_id(0)

    z = 1.0 / (1.0 + jnp.exp(-la_ref[...]))
    z = jnp.clip(z * (LIMIT_B - LIMIT_A) + LIMIT_A, 0.0, 1.0)
    w2 = ((1.0 / (1.0 + jnp.exp(-il_ref[...]))) * z)
    w2 = w2.astype(jnp.bfloat16)

    pT = jnp.transpose(p_ref[...]).astype(jnp.bfloat16)
    qT = jnp.transpose(q_ref[...]).astype(jnp.bfloat16)

    ki = lax.broadcasted_iota(jnp.int32, (R, R * C), 0)
    kj = lax.broadcasted_iota(jnp.int32, (R, R * C), 1)
    kred = (ki == kj // C).astype(jnp.bfloat16)
    sub8 = lax.broadcasted_iota(jnp.int32, (R, BR), 0)
    row_iota = lax.broadcasted_iota(jnp.int32, (1, BR), 1)
    valid = (step * BR + row_iota) < (E // 8)

    tot = jnp.zeros((), jnp.float32)
    for g in range(8):
        qtg = qT[g * C:(g + 1) * C, :]
        ptg = pT[g * C:(g + 1) * C, :]
        t2 = lax.dot_general(w2, qtg, (((1,), (0,)), ((), ())),
                             preferred_element_type=jnp.float32
                             ).astype(jnp.bfloat16)
        ptile = jnp.concatenate([ptg] * R, axis=0)
        u2 = t2 * ptile
        l8t = lax.dot_general(kred, u2, (((1,), (0,)), ((), ())),
                              preferred_element_type=jnp.float32)

        et_row = et_ref[0, g:g + 1, :]
        onehot = sub8 == et_row
        lsel = jnp.where(onehot, l8t, 0.0)
        bsel = jnp.where(onehot, b_ref[...], 0.0)
        logits = (jnp.sum(lsel, axis=0, keepdims=True)
                  + jnp.sum(bsel, axis=0, keepdims=True))

        sp = jnp.maximum(-logits, 0.0) + jnp.log(1.0 + jnp.exp(-jnp.abs(logits)))
        sp = jnp.where(valid, sp, 0.0)
        tot = tot + jnp.sum(sp)

    @pl.when(step == 0)
    def _init():
        out_ref[...] = jnp.zeros_like(out_ref)

    out_ref[...] = out_ref[...] + tot.reshape(1, 1)

    @pl.when(step == GSTEPS - 1)
    def _final():
        out_ref[...] = out_ref[...] * (1.0 / E)


def _tc_loss(p, q, et2, il2, la2, bias2):
    return pl.pallas_call(
        _tc_loss_body,
        grid=(GSTEPS,),
        in_specs=[
            pl.BlockSpec((BR, 128), lambda i: (i, 0)),
            pl.BlockSpec((BR, 128), lambda i: (i, 0)),
            pl.BlockSpec((1, R, BR), lambda i: (i, 0, 0)),
            pl.BlockSpec((R * C, C), lambda i: (0, 0)),
            pl.BlockSpec((R * C, C), lambda i: (0, 0)),
            pl.BlockSpec((R, 1), lambda i: (0, 0)),
        ],
        out_specs=pl.BlockSpec((1, 1), lambda i: (0, 0)),
        out_shape=jax.ShapeDtypeStruct((1, 1), jnp.float32),
        compiler_params=pltpu.CompilerParams(
            dimension_semantics=("arbitrary",)),
    )(p, q, et2, il2, la2, bias2)


def kernel(assignments, edge_index, edge_type, inter_cluster_logits,
           absent_bias, log_alpha):
    pad = E_PAD - E
    zpad = jnp.zeros((pad,), jnp.int32)
    src_p = jnp.concatenate([edge_index[0], zpad]).reshape(E_PAD // 128, 128)
    dst_p = jnp.concatenate([edge_index[1], zpad]).reshape(E_PAD // 128, 128)
    et2 = (jnp.concatenate([edge_type, zpad])
           .reshape(GSTEPS, BR, 8).transpose(0, 2, 1))
    il2 = inter_cluster_logits.reshape(R * C, C)
    la2 = log_alpha.reshape(R * C, C)
    bias2 = absent_bias.reshape(R, 1)

    p, q = _sc_gather(assignments, src_p, dst_p)
    pf = p.reshape(FROWS, 128)
    qf = q.reshape(FROWS, 128)
    out = _tc_loss(pf, qf, et2, il2, la2, bias2)
    return out[0, 0]


# speedup vs baseline: 1.3514x; 1.1451x over previous
import functools

import jax
import jax.numpy as jnp
from jax import lax
from jax.experimental import pallas as pl
from jax.experimental.pallas import tpu as pltpu
from jax.experimental.pallas import tpu_sc as plsc

N_NODES = 50000
E = 800000
R = 8
C = 16
LIMIT_A = -0.1
LIMIT_B = 1.1

NC = 2
NS = 16
NW = NC * NS

E_PAD = 819200
EDGES_PER_W = E_PAD // NW
CHUNK = 512
GROUPS = CHUNK // 128
N_CHUNKS = EDGES_PER_W // CHUNK
IDX_ROWS_PER_W = EDGES_PER_W // 128

BE = 32768
GSTEPS = E_PAD // BE
BR = BE // 8
FROWS = E_PAD * C // 128


def _sc_gather_body(table, sidx, didx, p_out, q_out,
                    sidx_all, didx_all,
                    srow0, srow1, drow0, drow1,
                    gs0, gs1, ws0, ws1):
    wid = lax.axis_index("s") * NC + lax.axis_index("c")
    row0 = wid * IDX_ROWS_PER_W
    base0 = wid * EDGES_PER_W
    srow_v = (srow0, srow1)
    drow_v = (drow0, drow1)
    gsem = (gs0, gs1)
    wsem = (ws0, ws1)

    pltpu.sync_copy(sidx.at[pl.ds(row0, IDX_ROWS_PER_W)], sidx_all)
    pltpu.sync_copy(didx.at[pl.ds(row0, IDX_ROWS_PER_W)], didx_all)

    def launch_g(i, b):
        for g in range(GROUPS):
            pltpu.async_copy(table.at[sidx_all.at[i * GROUPS + g]],
                             srow_v[b].at[pl.ds(g * 128, 128)], gsem[b])
            pltpu.async_copy(table.at[didx_all.at[i * GROUPS + g]],
                             drow_v[b].at[pl.ds(g * 128, 128)], gsem[b])

    def wait_g(b):
        pltpu.make_async_copy(p_out.at[pl.ds(0, CHUNK)], srow_v[b], gsem[b]).wait()
        pltpu.make_async_copy(p_out.at[pl.ds(0, CHUNK)], drow_v[b], gsem[b]).wait()

    def launch_wb(i, b):
        off = base0 + i * CHUNK
        pltpu.async_copy(srow_v[b], p_out.at[pl.ds(off, CHUNK)], wsem[b])
        pltpu.async_copy(drow_v[b], q_out.at[pl.ds(off, CHUNK)], wsem[b])

    def wait_wb(b):
        pltpu.make_async_copy(p_out.at[pl.ds(0, CHUNK)], srow_v[b], wsem[b]).wait()
        pltpu.make_async_copy(p_out.at[pl.ds(0, CHUNK)], drow_v[b], wsem[b]).wait()

    launch_g(0, 0)
    launch_g(1, 1)

    def body(k, carry):
        for b in (0, 1):
            i = 2 * k + b
            wait_g(b)
            launch_wb(i, b)
            wait_wb(b)
            launch_g(i + 2, b)
        return carry

    lax.fori_loop(0, N_CHUNKS // 2 - 1, body, 0)

    for b in (0, 1):
        wait_g(b)
        launch_wb(N_CHUNKS - 2 + b, b)
    wait_wb(0)
    wait_wb(1)


def _sc_gather(assignments, src_idx, dst_idx):
    kern = pl.kernel(
        _sc_gather_body,
        out_type=(jax.ShapeDtypeStruct((E_PAD, C), jnp.float32),
                  jax.ShapeDtypeStruct((E_PAD, C), jnp.float32)),
        mesh=plsc.VectorSubcoreMesh(core_axis_name="c", subcore_axis_name="s"),
        scratch_types=[
            pltpu.VMEM((IDX_ROWS_PER_W, 128), jnp.int32),
            pltpu.VMEM((IDX_ROWS_PER_W, 128), jnp.int32),
            pltpu.VMEM((CHUNK, C), jnp.float32),
            pltpu.VMEM((CHUNK, C), jnp.float32),
            pltpu.VMEM((CHUNK, C), jnp.float32),
            pltpu.VMEM((CHUNK, C), jnp.float32),
            pltpu.SemaphoreType.DMA,
            pltpu.SemaphoreType.DMA,
            pltpu.SemaphoreType.DMA,
            pltpu.SemaphoreType.DMA,
        ],
        compiler_params=pltpu.CompilerParams(use_tc_tiling_on_sc=False),
    )
    return kern(assignments, src_idx, dst_idx)


def _tc_loss_body(p_ref, q_ref, et_ref, il_ref, la_ref, b_ref, out_ref):
    step = pl.program_id(0)

    z = 1.0 / (1.0 + jnp.exp(-la_ref[...]))
    z = jnp.clip(z * (LIMIT_B - LIMIT_A) + LIMIT_A, 0.0, 1.0)
    w2 = ((1.0 / (1.0 + jnp.exp(-il_ref[...]))) * z)
    w2 = w2.astype(jnp.bfloat16)

    pT = jnp.transpose(p_ref[...]).astype(jnp.bfloat16)
    qT = jnp.transpose(q_ref[...]).astype(jnp.bfloat16)

    ki = lax.broadcasted_iota(jnp.int32, (R, R * C), 0)
    kj = lax.broadcasted_iota(jnp.int32, (R, R * C), 1)
    kred = (ki == kj // C).astype(jnp.bfloat16)
    sub8 = lax.broadcasted_iota(jnp.int32, (R, BR), 0)
    row_iota = lax.broadcasted_iota(jnp.int32, (1, BR), 1)
    valid = (step * BR + row_iota) < (E // 8)

    tot = jnp.zeros((), jnp.float32)
    for g in range(8):
        qtg = qT[g * C:(g + 1) * C, :]
        ptg = pT[g * C:(g + 1) * C, :]
        t2 = lax.dot_general(w2, qtg, (((1,), (0,)), ((), ())),
                             preferred_element_type=jnp.float32
                             ).astype(jnp.bfloat16)
        ptile = jnp.concatenate([ptg] * R, axis=0)
        u2 = t2 * ptile
        l8t = lax.dot_general(kred, u2, (((1,), (0,)), ((), ())),
                              preferred_element_type=jnp.float32)

        et_row = et_ref[0, g:g + 1, :].astype(jnp.int32)
        onehot = sub8 == et_row
        lsel = jnp.where(onehot, l8t, 0.0)
        bsel = jnp.where(onehot, b_ref[...], 0.0)
        logits = (jnp.sum(lsel, axis=0, keepdims=True)
                  + jnp.sum(bsel, axis=0, keepdims=True))

        sp = jnp.maximum(-logits, 0.0) + jnp.log(1.0 + jnp.exp(-jnp.abs(logits)))
        sp = jnp.where(valid, sp, 0.0)
        tot = tot + jnp.sum(sp)

    @pl.when(step == 0)
    def _init():
        out_ref[...] = jnp.zeros_like(out_ref)

    out_ref[...] = out_ref[...] + tot.reshape(1, 1)

    @pl.when(step == GSTEPS - 1)
    def _final():
        out_ref[...] = out_ref[...] * (1.0 / E)


def _tc_loss(p, q, et2, il2, la2, bias2):
    return pl.pallas_call(
        _tc_loss_body,
        grid=(GSTEPS,),
        in_specs=[
            pl.BlockSpec((BR, 128), lambda i: (i, 0)),
            pl.BlockSpec((BR, 128), lambda i: (i, 0)),
            pl.BlockSpec((1, R, BR), lambda i: (i, 0, 0)),
            pl.BlockSpec((R * C, C), lambda i: (0, 0)),
            pl.BlockSpec((R * C, C), lambda i: (0, 0)),
            pl.BlockSpec((R, 1), lambda i: (0, 0)),
        ],
        out_specs=pl.BlockSpec((1, 1), lambda i: (0, 0)),
        out_shape=jax.ShapeDtypeStruct((1, 1), jnp.float32),
        compiler_params=pltpu.CompilerParams(
            dimension_semantics=("arbitrary",)),
    )(p, q, et2, il2, la2, bias2)


def kernel(assignments, edge_index, edge_type, inter_cluster_logits,
           absent_bias, log_alpha):
    pad = E_PAD - E
    zpad = jnp.zeros((pad,), jnp.int32)
    src_p = jnp.concatenate([edge_index[0], zpad]).reshape(E_PAD // 128, 128)
    dst_p = jnp.concatenate([edge_index[1], zpad]).reshape(E_PAD // 128, 128)
    et2 = (jnp.concatenate([edge_type, zpad]).astype(jnp.int8)
           .reshape(GSTEPS, BR, 8).transpose(0, 2, 1))
    il2 = inter_cluster_logits.reshape(R * C, C)
    la2 = log_alpha.reshape(R * C, C)
    bias2 = absent_bias.reshape(R, 1)

    p, q = _sc_gather(assignments, src_p, dst_p)
    pf = p.reshape(FROWS, 128)
    qf = q.reshape(FROWS, 128)
    out = _tc_loss(pf, qf, et2, il2, la2, bias2)
    return out[0, 0]
